# baseline (speedup 1.0000x reference)
import functools

import jax
import jax.numpy as jnp
from jax import lax
from jax.experimental import pallas as pl
from jax.experimental.pallas import tpu as pltpu
from jax.experimental.pallas import tpu_sc as plsc

DIM = 128
MAX_DIS = 1000
B = 100000
C = 128
NW = 32
SPAN = 3200
NCH = SPAN // C
NBUF = 6
LOOKAHEAD = 4

_mesh = plsc.VectorSubcoreMesh(core_axis_name="c", subcore_axis_name="s")


@functools.partial(
    pl.kernel,
    mesh=_mesh,
    out_type=jax.ShapeDtypeStruct((B, DIM), jnp.float32),
    scratch_types=[
        pltpu.VMEM((SPAN,), jnp.int32),
        pltpu.VMEM((NBUF, C, DIM), jnp.float32),
        pltpu.VMEM_SHARED((2, MAX_DIS + 1, DIM), jnp.float32),
        pltpu.SemaphoreType.DMA((NBUF,)),
        pltpu.SemaphoreType.DMA((NBUF,)),
    ],
)
def _gather_kernel(d_hbm, embed_hbm, out_hbm, idx_v, rows_v, table_sh,
                   gsem, wsem):
    sid = lax.axis_index("s")
    wid = sid * 2 + lax.axis_index("c")
    base_w = jnp.minimum(wid * SPAN, B - SPAN)
    par = lax.rem(sid, 2)

    @pl.when(sid < 2)
    def _():
        pltpu.sync_copy(embed_hbm, table_sh.at[sid])

    pltpu.sync_copy(d_hbm.at[pl.ds(base_w, SPAN)], idx_v)

    def clamp_body(j, carry):
        sl = pl.ds(j * 16, 16)
        idx_v[sl] = jnp.minimum(jnp.maximum(idx_v[sl], 0), MAX_DIS)
        return carry

    lax.fori_loop(0, SPAN // 16, clamp_body, None)

    plsc.subcore_barrier()

    def gather_start(k, b):
        pltpu.make_async_copy(
            table_sh.at[par].at[idx_v.at[pl.ds(k * C, C)]],
            rows_v.at[b], gsem.at[b]
        ).start()

    def gather_wait(b):
        pltpu.make_async_copy(
            table_sh.at[par].at[idx_v.at[pl.ds(0, C)]],
            rows_v.at[b], gsem.at[b]
        ).wait()

    def write_start(k, b):
        pltpu.make_async_copy(
            rows_v.at[b], out_hbm.at[pl.ds(base_w + k * C, C)], wsem.at[b]
        ).start()

    def write_wait(b):
        pltpu.make_async_copy(
            rows_v.at[b], out_hbm.at[pl.ds(0, C)], wsem.at[b]
        ).wait()

    for k in range(LOOKAHEAD):
        gather_start(k, k % NBUF)

    def chunk_body(k, carry):
        kf = k + LOOKAHEAD

        @pl.when(kf < NCH)
        def _():
            bf = lax.rem(kf, NBUF)

            @pl.when(kf >= NBUF)
            def _():
                write_wait(bf)

            gather_start(kf, bf)

        b = lax.rem(k, NBUF)
        gather_wait(b)
        write_start(k, b)
        return carry

    lax.fori_loop(0, NCH, chunk_body, None)

    for b in range(NBUF):
        write_wait(b)


def kernel(d, embed_d):
    return _gather_kernel(d, embed_d)


# speedup vs baseline: 1.0088x; 1.0088x over previous
import functools

import jax
import jax.numpy as jnp
from jax import lax
from jax.experimental import pallas as pl
from jax.experimental.pallas import tpu as pltpu
from jax.experimental.pallas import tpu_sc as plsc

DIM = 128
MAX_DIS = 1000
B = 100000
C = 128
NW = 32
SPAN = 3200
NCH = SPAN // C
NBUF = 7
LOOKAHEAD = 5

_mesh = plsc.VectorSubcoreMesh(core_axis_name="c", subcore_axis_name="s")


@functools.partial(
    pl.kernel,
    mesh=_mesh,
    out_type=jax.ShapeDtypeStruct((B, DIM), jnp.float32),
    scratch_types=[
        pltpu.VMEM((SPAN,), jnp.int32),
        pltpu.VMEM((NBUF, C, DIM), jnp.float32),
        pltpu.VMEM_SHARED((MAX_DIS + 1, DIM), jnp.float32),
        pltpu.SemaphoreType.DMA((NBUF,)),
        pltpu.SemaphoreType.DMA((NBUF,)),
    ],
)
def _gather_kernel(d_hbm, embed_hbm, out_hbm, idx_v, rows_v, table_sh,
                   gsem, wsem):
    sid = lax.axis_index("s")
    wid = sid * 2 + lax.axis_index("c")
    base_w = jnp.minimum(wid * SPAN, B - SPAN)

    @pl.when(sid == 0)
    def _():
        pltpu.sync_copy(embed_hbm, table_sh)

    pltpu.sync_copy(d_hbm.at[pl.ds(base_w, SPAN)], idx_v)

    def clamp_body(j, carry):
        sl = pl.ds(j * 16, 16)
        idx_v[sl] = jnp.minimum(jnp.maximum(idx_v[sl], 0), MAX_DIS)
        return carry

    lax.fori_loop(0, SPAN // 16, clamp_body, None)

    plsc.subcore_barrier()

    def gather_start(k, b):
        pltpu.make_async_copy(
            table_sh.at[idx_v.at[pl.ds(k * C, C)]], rows_v.at[b], gsem.at[b]
        ).start()

    def gather_wait(b):
        pltpu.make_async_copy(
            table_sh.at[idx_v.at[pl.ds(0, C)]], rows_v.at[b], gsem.at[b]
        ).wait()

    def write_start(k, b):
        pltpu.make_async_copy(
            rows_v.at[b], out_hbm.at[pl.ds(base_w + k * C, C)], wsem.at[b]
        ).start()

    def write_wait(b):
        pltpu.make_async_copy(
            rows_v.at[b], out_hbm.at[pl.ds(0, C)], wsem.at[b]
        ).wait()

    for k in range(LOOKAHEAD):
        gather_start(k, k % NBUF)

    def chunk_body(k, carry):
        kf = k + LOOKAHEAD

        @pl.when(kf < NCH)
        def _():
            bf = lax.rem(kf, NBUF)

            @pl.when(kf >= NBUF)
            def _():
                write_wait(bf)

            gather_start(kf, bf)

        b = lax.rem(k, NBUF)
        gather_wait(b)
        write_start(k, b)
        return carry

    lax.fori_loop(0, NCH, chunk_body, None)

    for b in range(NBUF):
        write_wait(b)


def kernel(d, embed_d):
    return _gather_kernel(d, embed_d)


# speedup vs baseline: 1.0169x; 1.0081x over previous
import functools

import jax
import jax.numpy as jnp
from jax import lax
from jax.experimental import pallas as pl
from jax.experimental.pallas import tpu as pltpu
from jax.experimental.pallas import tpu_sc as plsc

DIM = 128
MAX_DIS = 1000
B = 100000
C = 64
NW = 32
SPAN = 3200
NCH = SPAN // C
NBUF = 12
LOOKAHEAD = 10

_mesh = plsc.VectorSubcoreMesh(core_axis_name="c", subcore_axis_name="s")


@functools.partial(
    pl.kernel,
    mesh=_mesh,
    out_type=jax.ShapeDtypeStruct((B, DIM), jnp.float32),
    scratch_types=[
        pltpu.VMEM((SPAN,), jnp.int32),
        pltpu.VMEM((NBUF, C, DIM), jnp.float32),
        pltpu.VMEM_SHARED((MAX_DIS + 1, DIM), jnp.float32),
        pltpu.SemaphoreType.DMA((NBUF,)),
        pltpu.SemaphoreType.DMA((NBUF,)),
    ],
)
def _gather_kernel(d_hbm, embed_hbm, out_hbm, idx_v, rows_v, table_sh,
                   gsem, wsem):
    sid = lax.axis_index("s")
    wid = sid * 2 + lax.axis_index("c")
    base_w = jnp.minimum(wid * SPAN, B - SPAN)

    @pl.when(sid == 0)
    def _():
        pltpu.sync_copy(embed_hbm, table_sh)

    pltpu.sync_copy(d_hbm.at[pl.ds(base_w, SPAN)], idx_v)

    def clamp_body(j, carry):
        sl = pl.ds(j * 16, 16)
        idx_v[sl] = jnp.minimum(jnp.maximum(idx_v[sl], 0), MAX_DIS)
        return carry

    lax.fori_loop(0, SPAN // 16, clamp_body, None)

    plsc.subcore_barrier()

    def gather_start(k, b):
        pltpu.make_async_copy(
            table_sh.at[idx_v.at[pl.ds(k * C, C)]], rows_v.at[b], gsem.at[b]
        ).start()

    def gather_wait(b):
        pltpu.make_async_copy(
            table_sh.at[idx_v.at[pl.ds(0, C)]], rows_v.at[b], gsem.at[b]
        ).wait()

    def write_start(k, b):
        pltpu.make_async_copy(
            rows_v.at[b], out_hbm.at[pl.ds(base_w + k * C, C)], wsem.at[b]
        ).start()

    def write_wait(b):
        pltpu.make_async_copy(
            rows_v.at[b], out_hbm.at[pl.ds(0, C)], wsem.at[b]
        ).wait()

    for k in range(LOOKAHEAD):
        gather_start(k, k % NBUF)

    def chunk_body(k, carry):
        kf = k + LOOKAHEAD

        @pl.when(kf < NCH)
        def _():
            bf = lax.rem(kf, NBUF)

            @pl.when(kf >= NBUF)
            def _():
                write_wait(bf)

            gather_start(kf, bf)

        b = lax.rem(k, NBUF)
        gather_wait(b)
        write_start(k, b)
        return carry

    lax.fori_loop(0, NCH, chunk_body, None)

    for b in range(NBUF):
        write_wait(b)


def kernel(d, embed_d):
    return _gather_kernel(d, embed_d)


# speedup vs baseline: 1.0177x; 1.0008x over previous
import functools

import jax
import jax.numpy as jnp
from jax import lax
from jax.experimental import pallas as pl
from jax.experimental.pallas import tpu as pltpu
from jax.experimental.pallas import tpu_sc as plsc

DIM = 128
MAX_DIS = 1000
B = 100000
C = 64
NW = 32
SPAN = 3200
NCH = SPAN // C
NBUF = 12
LOOKAHEAD = 8

_mesh = plsc.VectorSubcoreMesh(core_axis_name="c", subcore_axis_name="s")


@functools.partial(
    pl.kernel,
    mesh=_mesh,
    out_type=jax.ShapeDtypeStruct((B, DIM), jnp.float32),
    scratch_types=[
        pltpu.VMEM((SPAN,), jnp.int32),
        pltpu.VMEM((NBUF, C, DIM), jnp.float32),
        pltpu.VMEM_SHARED((MAX_DIS + 1, DIM), jnp.float32),
        pltpu.SemaphoreType.DMA((NBUF,)),
        pltpu.SemaphoreType.DMA((NBUF,)),
    ],
)
def _gather_kernel(d_hbm, embed_hbm, out_hbm, idx_v, rows_v, table_sh,
                   gsem, wsem):
    sid = lax.axis_index("s")
    wid = sid * 2 + lax.axis_index("c")
    base_w = jnp.minimum(wid * SPAN, B - SPAN)

    @pl.when(sid == 0)
    def _():
        pltpu.sync_copy(embed_hbm, table_sh)

    pltpu.sync_copy(d_hbm.at[pl.ds(base_w, SPAN)], idx_v)

    def clamp_body(j, carry):
        sl = pl.ds(j * 16, 16)
        idx_v[sl] = jnp.minimum(jnp.maximum(idx_v[sl], 0), MAX_DIS)
        return carry

    lax.fori_loop(0, SPAN // 16, clamp_body, None)

    plsc.subcore_barrier()

    def gather_start(k, b):
        pltpu.make_async_copy(
            table_sh.at[idx_v.at[pl.ds(k * C, C)]], rows_v.at[b], gsem.at[b]
        ).start()

    def gather_wait(b):
        pltpu.make_async_copy(
            table_sh.at[idx_v.at[pl.ds(0, C)]], rows_v.at[b], gsem.at[b]
        ).wait()

    def write_start(k, b):
        pltpu.make_async_copy(
            rows_v.at[b], out_hbm.at[pl.ds(base_w + k * C, C)], wsem.at[b]
        ).start()

    def write_wait(b):
        pltpu.make_async_copy(
            rows_v.at[b], out_hbm.at[pl.ds(0, C)], wsem.at[b]
        ).wait()

    for k in range(LOOKAHEAD):
        gather_start(k, k % NBUF)

    def chunk_body(k, carry):
        kf = k + LOOKAHEAD

        @pl.when(kf < NCH)
        def _():
            bf = lax.rem(kf, NBUF)

            @pl.when(kf >= NBUF)
            def _():
                write_wait(bf)

            gather_start(kf, bf)

        b = lax.rem(k, NBUF)
        gather_wait(b)
        write_start(k, b)
        return carry

    lax.fori_loop(0, NCH, chunk_body, None)

    for b in range(NBUF):
        write_wait(b)


def kernel(d, embed_d):
    return _gather_kernel(d, embed_d)


# speedup vs baseline: 1.0541x; 1.0358x over previous
import functools

import jax
import jax.numpy as jnp
from jax import lax
from jax.experimental import pallas as pl
from jax.experimental.pallas import tpu as pltpu
from jax.experimental.pallas import tpu_sc as plsc

DIM = 128
MAX_DIS = 1000
B = 100000
C = 64
NW = 32
SPAN = 3200
NCH = SPAN // C
NBUF = 12
LOOKAHEAD = 8

_mesh = plsc.VectorSubcoreMesh(core_axis_name="c", subcore_axis_name="s")


@functools.partial(
    pl.kernel,
    mesh=_mesh,
    out_type=jax.ShapeDtypeStruct((B, DIM), jnp.float32),
    scratch_types=[
        pltpu.VMEM((SPAN,), jnp.int32),
        pltpu.VMEM((NBUF, C, DIM), jnp.float32),
        pltpu.VMEM_SHARED((MAX_DIS + 1, DIM), jnp.float32),
        pltpu.SemaphoreType.DMA((NBUF,)),
        pltpu.SemaphoreType.DMA((NBUF,)),
    ],
)
def _gather_kernel(d_hbm, embed_hbm, out_hbm, idx_v, rows_v, table_sh,
                   gsem, wsem):
    sid = lax.axis_index("s")
    wid = sid * 2 + lax.axis_index("c")
    base_w = jnp.minimum(wid * SPAN, B - SPAN)

    @pl.when(sid == 0)
    def _():
        pltpu.make_async_copy(embed_hbm, table_sh, gsem.at[0]).start()

    pltpu.sync_copy(d_hbm.at[pl.ds(base_w, SPAN)], idx_v)

    def clamp_chunk(k):
        for j in range(C // 16):
            sl = pl.ds(k * C + j * 16, 16)
            idx_v[sl] = jnp.minimum(jnp.maximum(idx_v[sl], 0), MAX_DIS)

    for k in range(LOOKAHEAD):
        clamp_chunk(k)

    @pl.when(sid == 0)
    def _():
        pltpu.make_async_copy(embed_hbm, table_sh, gsem.at[0]).wait()

    plsc.subcore_barrier()

    def gather_start(k, b):
        pltpu.make_async_copy(
            table_sh.at[idx_v.at[pl.ds(k * C, C)]], rows_v.at[b], gsem.at[b]
        ).start()

    def gather_wait(b):
        pltpu.make_async_copy(
            table_sh.at[idx_v.at[pl.ds(0, C)]], rows_v.at[b], gsem.at[b]
        ).wait()

    def write_start(k, b):
        pltpu.make_async_copy(
            rows_v.at[b], out_hbm.at[pl.ds(base_w + k * C, C)], wsem.at[b]
        ).start()

    def write_wait(b):
        pltpu.make_async_copy(
            rows_v.at[b], out_hbm.at[pl.ds(0, C)], wsem.at[b]
        ).wait()

    for k in range(LOOKAHEAD):
        gather_start(k, k % NBUF)

    def chunk_body(k, carry):
        kf = k + LOOKAHEAD

        @pl.when(kf < NCH)
        def _():
            bf = lax.rem(kf, NBUF)

            @pl.when(kf >= NBUF)
            def _():
                write_wait(bf)

            clamp_chunk(kf)
            gather_start(kf, bf)

        b = lax.rem(k, NBUF)
        gather_wait(b)
        write_start(k, b)
        return carry

    lax.fori_loop(0, NCH, chunk_body, None)

    for b in range(NBUF):
        write_wait(b)


def kernel(d, embed_d):
    return _gather_kernel(d, embed_d)


# speedup vs baseline: 1.0573x; 1.0030x over previous
import functools

import jax
import jax.numpy as jnp
from jax import lax
from jax.experimental import pallas as pl
from jax.experimental.pallas import tpu as pltpu
from jax.experimental.pallas import tpu_sc as plsc

DIM = 128
MAX_DIS = 1000
B = 100000
C = 64
NW = 32
SPAN = 3200
NCH = SPAN // C
NBUF = 12
LOOKAHEAD = 8

_mesh = plsc.VectorSubcoreMesh(core_axis_name="c", subcore_axis_name="s")


@functools.partial(
    pl.kernel,
    mesh=_mesh,
    out_type=jax.ShapeDtypeStruct((B, DIM), jnp.float32),
    scratch_types=[
        pltpu.VMEM((SPAN,), jnp.int32),
        pltpu.VMEM((NBUF, C, DIM), jnp.float32),
        pltpu.VMEM_SHARED((MAX_DIS + 1, DIM), jnp.float32),
        pltpu.SemaphoreType.DMA((NBUF,)),
        pltpu.SemaphoreType.DMA((NBUF,)),
    ],
)
def _gather_kernel(d_hbm, embed_hbm, out_hbm, idx_v, rows_v, table_sh,
                   gsem, wsem):
    sid = lax.axis_index("s")
    wid = sid * 2 + lax.axis_index("c")
    base_w = jnp.minimum(wid * SPAN, B - SPAN)

    @pl.when(sid == 0)
    def _():
        pltpu.make_async_copy(embed_hbm, table_sh, gsem.at[0]).start()

    HEAD = LOOKAHEAD * C
    pltpu.sync_copy(d_hbm.at[pl.ds(base_w, HEAD)], idx_v.at[pl.ds(0, HEAD)])
    pltpu.make_async_copy(
        d_hbm.at[pl.ds(base_w + HEAD, SPAN - HEAD)],
        idx_v.at[pl.ds(HEAD, SPAN - HEAD)], wsem.at[0]
    ).start()

    def clamp_chunk(k):
        for j in range(C // 16):
            sl = pl.ds(k * C + j * 16, 16)
            idx_v[sl] = jnp.minimum(jnp.maximum(idx_v[sl], 0), MAX_DIS)

    for k in range(LOOKAHEAD):
        clamp_chunk(k)

    @pl.when(sid == 0)
    def _():
        pltpu.make_async_copy(embed_hbm, table_sh, gsem.at[0]).wait()

    plsc.subcore_barrier()

    def gather_start(k, b):
        pltpu.make_async_copy(
            table_sh.at[idx_v.at[pl.ds(k * C, C)]], rows_v.at[b], gsem.at[b]
        ).start()

    def gather_wait(b):
        pltpu.make_async_copy(
            table_sh.at[idx_v.at[pl.ds(0, C)]], rows_v.at[b], gsem.at[b]
        ).wait()

    def write_start(k, b):
        pltpu.make_async_copy(
            rows_v.at[b], out_hbm.at[pl.ds(base_w + k * C, C)], wsem.at[b]
        ).start()

    def write_wait(b):
        pltpu.make_async_copy(
            rows_v.at[b], out_hbm.at[pl.ds(0, C)], wsem.at[b]
        ).wait()

    for k in range(LOOKAHEAD):
        gather_start(k, k % NBUF)

    pltpu.make_async_copy(
        d_hbm.at[pl.ds(base_w + HEAD, SPAN - HEAD)],
        idx_v.at[pl.ds(HEAD, SPAN - HEAD)], wsem.at[0]
    ).wait()

    def chunk_body(k, carry):
        kf = k + LOOKAHEAD

        @pl.when(kf < NCH)
        def _():
            bf = lax.rem(kf, NBUF)

            @pl.when(kf >= NBUF)
            def _():
                write_wait(bf)

            clamp_chunk(kf)
            gather_start(kf, bf)

        b = lax.rem(k, NBUF)
        gather_wait(b)
        write_start(k, b)
        return carry

    lax.fori_loop(0, NCH, chunk_body, None)

    for b in range(NBUF):
        write_wait(b)


def kernel(d, embed_d):
    return _gather_kernel(d, embed_d)
